# baseline (speedup 1.0000x reference)
import functools

import jax
import jax.numpy as jnp
from jax import lax
from jax.experimental import pallas as pl
from jax.experimental.pallas import tpu as pltpu
from jax.experimental.pallas import tpu_sc as plsc

BATCH = 16384
DIM = 64
VOCAB = 1000000
MAIN_V = 999424
STRIPE = MAIN_V // 32
TAIL = VOCAB - MAIN_V
CH_LANES = 256
N_CH = STRIPE // CH_LANES
N_GRP = BATCH // 16

_info = plsc.get_sparse_core_info()
_NC, _NS = _info.num_cores, _info.num_subcores

_mesh = plsc.VectorSubcoreMesh(core_axis_name="c", subcore_axis_name="s")


@functools.partial(
    pl.kernel,
    mesh=_mesh,
    out_type=jax.ShapeDtypeStruct((BATCH, 128), jnp.float32),
    scratch_types=[
        pltpu.VMEM((BATCH + 16,), jnp.int32),
        pltpu.VMEM((BATCH + 16,), jnp.int32),
        pltpu.VMEM((2, 8, 8, CH_LANES), jnp.float32),
        pltpu.VMEM((64, TAIL), jnp.float32),
        pltpu.VMEM((32, 128), jnp.float32),
        pltpu.VMEM((32,), jnp.int32),
        pltpu.SemaphoreType.DMA,
        pltpu.SemaphoreType.DMA,
        pltpu.SemaphoreType.DMA,
        pltpu.SemaphoreType.DMA,
    ],
    compiler_params=pltpu.CompilerParams(needs_layout_passes=False),
)
def _embed(idx_hbm, wt3_hbm, wtail_hbm, out_hbm, idx_v, hits_v, buf_v,
           wtail_v, ext_v, bext_v, st_sem0, st_sem1, sc_sem0, sc_sem1):
    wid = lax.axis_index("s") * _NC + lax.axis_index("c")
    lo = wid * STRIPE
    iot = lax.iota(jnp.int32, 16)
    lane0 = iot == 0

    pltpu.sync_copy(idx_hbm.at[pl.ds(0, BATCH)], idx_v.at[pl.ds(0, BATCH)])
    pltpu.sync_copy(wtail_hbm, wtail_v)

    def prepass(q, cur):
        iv = idx_v[pl.ds(q * 16, 16)]
        bv = q * 16 + iot
        m_main = (iv >= lo) & (iv < lo + STRIPE)
        m_tail = (iv >= MAIN_V) & ((bv & 31) == wid)
        rel = jnp.where(m_tail, iv - MAIN_V + STRIPE, iv - lo)
        m = m_main | m_tail
        packed = (rel << 14) | bv
        mi = jnp.where(m, 1, 0)
        csum = jnp.cumsum(mi)
        pos = cur + csum - mi
        plsc.store_scatter(hits_v, [pos], packed, mask=m)
        return cur + csum[15]

    n_hits = lax.fori_loop(0, N_GRP, prepass, 0)
    n_hit_grp = (n_hits + 15) >> 4

    sems = (st_sem0, st_sem1)

    def fire_window(t, p):
        off = lo + t * CH_LANES
        for g in range(8):
            pltpu.async_copy(
                wt3_hbm.at[g, :, pl.ds(off, CH_LANES)],
                buf_v.at[p, g],
                sems[p],
            )

    def wait_window(p):
        for g in range(8):
            pltpu.make_async_copy(
                wt3_hbm.at[0, :, pl.ds(0, CH_LANES)],
                buf_v.at[p, 0],
                sems[p],
            ).wait()

    def scan_window(t, p, state, tail=False):
        rlo = STRIPE if tail else t * CH_LANES
        wlen = TAIL if tail else CH_LANES

        def grp(q, st):
            slot, f0, f1 = st
            hv = hits_v[pl.ds(q * 16, 16)]
            rel = lax.shift_right_logical(hv, 14)
            bv = hv & 16383
            m = (rel >= rlo) & (rel < rlo + wlen) & ((q * 16 + iot) < n_hits)
            mi = jnp.where(m, 1, 0)
            for l in range(16):
                hit = mi[l] == 1
                lrel = rel[l] - rlo
                lrel_v = iot * 0 + lrel
                row_v = iot * 0 + (slot & 31)

                @pl.when(hit)
                def _():
                    for jq in range(4):
                        if tail:
                            vals = plsc.load_gather(
                                wtail_v, [jq * 16 + iot, lrel_v]
                            )
                        else:
                            vals = plsc.load_gather(
                                buf_v.at[p],
                                [2 * jq + (iot >> 3), iot & 7, lrel_v],
                            )
                        plsc.store_scatter(
                            ext_v, [row_v, jq * 16 + iot], vals
                        )
                    plsc.store_scatter(
                        bext_v, [row_v], iot * 0 + bv[l], mask=lane0
                    )

                slot_new = slot + mi[l]
                fl = hit & ((slot_new & 15) == 0)
                half = (slot >> 4) & 1

                @pl.when(fl & (half == 0))
                def _():
                    _flush(0, f0, sc_sem0)

                @pl.when(fl & (half == 1))
                def _():
                    _flush(1, f1, sc_sem1)

                f0 = jnp.where(fl & (half == 0), 1, f0)
                f1 = jnp.where(fl & (half == 1), 1, f1)
                slot = slot_new & 31
            return slot, f0, f1

        return lax.fori_loop(0, n_hit_grp, grp, state)

    def _flush(h, pending, sem):
        bv = bext_v[pl.ds(h * 16, 16)]

        @pl.when(pending == 1)
        def _():
            pltpu.make_async_copy(
                ext_v.at[pl.ds(h * 16, 16)], out_hbm.at[bv], sem
            ).wait()

        pltpu.async_copy(
            ext_v.at[pl.ds(h * 16, 16)], out_hbm.at[bv], sem
        )

    fire_window(0, 0)
    fire_window(1, 1)

    def outer(k, state):
        for p in range(2):
            t = 2 * k + p
            wait_window(p)
            state = scan_window(t, p, state)

            @pl.when(t + 2 < N_CH)
            def _():
                fire_window(t + 2, p)

        return state

    state = lax.fori_loop(0, N_CH // 2, outer, (0, 0, 0))
    slot, f0, f1 = scan_window(N_CH, 0, state, tail=True)

    rem = slot & 15
    src_row = (slot - 1) & 31
    src_v = iot * 0 + src_row
    pb = plsc.load_gather(bext_v, [src_v])

    for r in range(1, 16):
        @pl.when((rem != 0) & (r >= rem))
        def _():
            row_v = iot * 0 + ((slot & 16) + r)
            for jq in range(8):
                vals = plsc.load_gather(ext_v, [src_v, jq * 16 + iot])
                plsc.store_scatter(ext_v, [row_v, jq * 16 + iot], vals)
            plsc.store_scatter(bext_v, [row_v], pb, mask=lane0)

    half = (slot >> 4) & 1

    @pl.when((rem != 0) & (half == 0))
    def _():
        _flush(0, f0, sc_sem0)

    @pl.when((rem != 0) & (half == 1))
    def _():
        _flush(1, f1, sc_sem1)

    f0 = jnp.where((rem != 0) & (half == 0), 1, f0)
    f1 = jnp.where((rem != 0) & (half == 1), 1, f1)

    @pl.when(f0 == 1)
    def _():
        bv = bext_v[pl.ds(0, 16)]
        pltpu.make_async_copy(
            ext_v.at[pl.ds(0, 16)], out_hbm.at[bv], sc_sem0
        ).wait()

    @pl.when(f1 == 1)
    def _():
        bv = bext_v[pl.ds(16, 16)]
        pltpu.make_async_copy(
            ext_v.at[pl.ds(16, 16)], out_hbm.at[bv], sc_sem1
        ).wait()


def kernel(x, weight):
    wt3 = weight.T.reshape(8, 8, VOCAB)
    wtail = weight[MAIN_V:].T
    out128 = _embed(x.astype(jnp.int32), wt3, wtail)
    return out128[:, :DIM]


# speedup vs baseline: 9.0531x; 9.0531x over previous
import functools

import jax
import jax.numpy as jnp
from jax import lax
from jax.experimental import pallas as pl
from jax.experimental.pallas import tpu as pltpu
from jax.experimental.pallas import tpu_sc as plsc

BATCH = 16384
DIM = 64
VOCAB = 1000000
MAIN_V = 999424
STRIPE = MAIN_V // 32
TAIL = VOCAB - MAIN_V
CH_LANES = 256
N_CH = STRIPE // CH_LANES
N_GRP = BATCH // 16

_info = plsc.get_sparse_core_info()
_NC, _NS = _info.num_cores, _info.num_subcores

_mesh = plsc.VectorSubcoreMesh(core_axis_name="c", subcore_axis_name="s")


@functools.partial(
    pl.kernel,
    mesh=_mesh,
    out_type=jax.ShapeDtypeStruct((BATCH, 128), jnp.float32),
    scratch_types=[
        pltpu.VMEM((BATCH + 16,), jnp.int32),
        pltpu.VMEM((BATCH + 16,), jnp.int32),
        pltpu.VMEM((2, 8, 8, CH_LANES), jnp.float32),
        pltpu.VMEM((64, TAIL), jnp.float32),
        pltpu.VMEM((32, 128), jnp.float32),
        pltpu.VMEM((32,), jnp.int32),
        pltpu.SemaphoreType.DMA,
        pltpu.SemaphoreType.DMA,
        pltpu.SemaphoreType.DMA,
        pltpu.SemaphoreType.DMA,
    ],
    compiler_params=pltpu.CompilerParams(needs_layout_passes=False),
)
def _embed(idx_hbm, wt3_hbm, wtail_hbm, out_hbm, idx_v, hits_v, buf_v,
           wtail_v, ext_v, bext_v, st_sem0, st_sem1, sc_sem0, sc_sem1):
    wid = lax.axis_index("s") * _NC + lax.axis_index("c")
    lo = wid * STRIPE
    iot = lax.iota(jnp.int32, 16)
    lane0 = iot == 0

    pltpu.sync_copy(idx_hbm.at[pl.ds(0, BATCH)], idx_v.at[pl.ds(0, BATCH)])
    pltpu.sync_copy(wtail_hbm, wtail_v)

    def prepass(q, cur):
        iv = idx_v[pl.ds(q * 16, 16)]
        bv = q * 16 + iot
        m_main = (iv >= lo) & (iv < lo + STRIPE)
        m_tail = (iv >= MAIN_V) & ((bv & 31) == wid)
        rel = jnp.where(m_tail, iv - MAIN_V + STRIPE, iv - lo)
        m = m_main | m_tail
        packed = (rel << 14) | bv
        mi = jnp.where(m, 1, 0)
        csum = jnp.cumsum(mi)
        pos = cur + csum - mi
        plsc.store_scatter(hits_v, [pos], packed, mask=m)
        return cur + csum[15]

    n_hits = lax.fori_loop(0, N_GRP, prepass, 0)
    n_hit_grp = (n_hits + 15) >> 4

    sems = (st_sem0, st_sem1)

    def fire_window(t, p):
        off = lo + t * CH_LANES
        for g in range(8):
            pltpu.async_copy(
                wt3_hbm.at[g, :, pl.ds(off, CH_LANES)],
                buf_v.at[p, g],
                sems[p],
            )

    def wait_window(p):
        for g in range(8):
            pltpu.make_async_copy(
                wt3_hbm.at[0, :, pl.ds(0, CH_LANES)],
                buf_v.at[p, 0],
                sems[p],
            ).wait()

    def scan_window(t, p, state, tail=False):
        rlo = STRIPE if tail else t * CH_LANES
        wlen = TAIL if tail else CH_LANES

        def grp(q, st):
            slot, f0, f1 = st
            hv = hits_v[pl.ds(q * 16, 16)]
            rel = lax.shift_right_logical(hv, 14)
            bv = hv & 16383
            m = (rel >= rlo) & (rel < rlo + wlen) & ((q * 16 + iot) < n_hits)
            mi = jnp.where(m, 1, 0)
            cnt = plsc.all_reduce_population_count(m)[0]

            @pl.when(cnt > 0)
            def _():
                pre = jnp.cumsum(mi) - mi
                for l in range(16):
                    @pl.when(mi[l] == 1)
                    def _():
                        slot_l = (slot + pre[l]) & 31
                        row_v = iot * 0 + slot_l
                        lrel_v = iot * 0 + (rel[l] - rlo)
                        for jq in range(4):
                            if tail:
                                vals = plsc.load_gather(
                                    wtail_v, [jq * 16 + iot, lrel_v]
                                )
                            else:
                                vals = plsc.load_gather(
                                    buf_v.at[p],
                                    [2 * jq + (iot >> 3), iot & 7, lrel_v],
                                )
                            plsc.store_scatter(
                                ext_v, [row_v, jq * 16 + iot], vals
                            )
                        plsc.store_scatter(
                            bext_v, [row_v], iot * 0 + bv[l], mask=lane0
                        )
                        fills = (slot_l & 15) == 15
                        half_l = (slot_l >> 4) & 1

                        @pl.when(fills & (half_l == 0))
                        def _():
                            _flush(0, f0, sc_sem0)

                        @pl.when(fills & (half_l == 1))
                        def _():
                            _flush(1, f1, sc_sem1)

            crossed = ((slot & 15) + cnt) >= 16
            half_c = (slot >> 4) & 1
            f0 = jnp.where(crossed & (half_c == 0), 1, f0)
            f1 = jnp.where(crossed & (half_c == 1), 1, f1)
            slot = (slot + cnt) & 31
            return slot, f0, f1

        return lax.fori_loop(0, n_hit_grp, grp, state)

    def _flush(h, pending, sem):
        bv = bext_v[pl.ds(h * 16, 16)]

        @pl.when(pending == 1)
        def _():
            pltpu.make_async_copy(
                ext_v.at[pl.ds(h * 16, 16)], out_hbm.at[bv], sem
            ).wait()

        pltpu.async_copy(
            ext_v.at[pl.ds(h * 16, 16)], out_hbm.at[bv], sem
        )

    fire_window(0, 0)
    fire_window(1, 1)

    def outer(k, state):
        for p in range(2):
            t = 2 * k + p
            wait_window(p)
            state = scan_window(t, p, state)

            @pl.when(t + 2 < N_CH)
            def _():
                fire_window(t + 2, p)

        return state

    state = lax.fori_loop(0, N_CH // 2, outer, (0, 0, 0))
    slot, f0, f1 = scan_window(N_CH, 0, state, tail=True)

    rem = slot & 15
    src_row = (slot - 1) & 31
    src_v = iot * 0 + src_row
    pb = plsc.load_gather(bext_v, [src_v])

    for r in range(1, 16):
        @pl.when((rem != 0) & (r >= rem))
        def _():
            row_v = iot * 0 + ((slot & 16) + r)
            for jq in range(8):
                vals = plsc.load_gather(ext_v, [src_v, jq * 16 + iot])
                plsc.store_scatter(ext_v, [row_v, jq * 16 + iot], vals)
            plsc.store_scatter(bext_v, [row_v], pb, mask=lane0)

    half = (slot >> 4) & 1

    @pl.when((rem != 0) & (half == 0))
    def _():
        _flush(0, f0, sc_sem0)

    @pl.when((rem != 0) & (half == 1))
    def _():
        _flush(1, f1, sc_sem1)

    f0 = jnp.where((rem != 0) & (half == 0), 1, f0)
    f1 = jnp.where((rem != 0) & (half == 1), 1, f1)

    @pl.when(f0 == 1)
    def _():
        bv = bext_v[pl.ds(0, 16)]
        pltpu.make_async_copy(
            ext_v.at[pl.ds(0, 16)], out_hbm.at[bv], sc_sem0
        ).wait()

    @pl.when(f1 == 1)
    def _():
        bv = bext_v[pl.ds(16, 16)]
        pltpu.make_async_copy(
            ext_v.at[pl.ds(16, 16)], out_hbm.at[bv], sc_sem1
        ).wait()


def kernel(x, weight):
    wt3 = weight.T.reshape(8, 8, VOCAB)
    wtail = weight[MAIN_V:].T
    out128 = _embed(x.astype(jnp.int32), wt3, wtail)
    return out128[:, :DIM]


# speedup vs baseline: 12.5484x; 1.3861x over previous
import functools

import jax
import jax.numpy as jnp
from jax import lax
from jax.experimental import pallas as pl
from jax.experimental.pallas import tpu as pltpu
from jax.experimental.pallas import tpu_sc as plsc

BATCH = 16384
DIM = 64
VOCAB = 1000000
MAIN_V = 999424
STRIPE = MAIN_V // 32
TAIL = VOCAB - MAIN_V
CH = 512
N_MAIN = STRIPE // CH
N_WIN = N_MAIN + 2
N_GRP = BATCH // 16

_info = plsc.get_sparse_core_info()
_NC, _NS = _info.num_cores, _info.num_subcores

_mesh = plsc.VectorSubcoreMesh(core_axis_name="c", subcore_axis_name="s")


@functools.partial(
    pl.kernel,
    mesh=_mesh,
    out_type=jax.ShapeDtypeStruct((BATCH, 128), jnp.float32),
    scratch_types=[
        pltpu.VMEM((BATCH + 16,), jnp.int32),
        pltpu.VMEM((BATCH + 16,), jnp.int32),
        pltpu.VMEM((2, 8, 8, CH), jnp.float32),
        pltpu.VMEM((32, 128), jnp.float32),
        pltpu.VMEM((32,), jnp.int32),
        pltpu.SemaphoreType.DMA,
        pltpu.SemaphoreType.DMA,
        pltpu.SemaphoreType.DMA,
        pltpu.SemaphoreType.DMA,
    ],
    compiler_params=pltpu.CompilerParams(needs_layout_passes=False),
)
def _embed(idx_hbm, wt3_hbm, wtl3_hbm, out_hbm, idx_v, hits_v, buf_v,
           ext_v, bext_v, st_sem0, st_sem1, sc_sem0, sc_sem1):
    wid = lax.axis_index("s") * _NC + lax.axis_index("c")
    lo = wid * STRIPE
    iot = lax.iota(jnp.int32, 16)
    lane0 = iot == 0

    pltpu.sync_copy(idx_hbm.at[pl.ds(0, BATCH)], idx_v.at[pl.ds(0, BATCH)])

    def prepass(q, cur):
        iv = idx_v[pl.ds(q * 16, 16)]
        bv = q * 16 + iot
        m_main = (iv >= lo) & (iv < lo + STRIPE)
        m_tail = (iv >= MAIN_V) & ((bv & 31) == wid)
        rel = jnp.where(m_tail, iv - MAIN_V + STRIPE, iv - lo)
        m = m_main | m_tail
        packed = (rel << 14) | bv
        mi = jnp.where(m, 1, 0)
        csum = jnp.cumsum(mi)
        pos = cur + csum - mi
        plsc.store_scatter(hits_v, [pos], packed, mask=m)
        return cur + csum[15]

    n_hits = lax.fori_loop(0, N_GRP, prepass, 0)
    n_hit_grp = (n_hits + 15) >> 4

    sems = (st_sem0, st_sem1)

    def fire_window(t, p):
        @pl.when(t < N_MAIN)
        def _():
            off = lo + t * CH
            for g in range(8):
                pltpu.async_copy(
                    wt3_hbm.at[g, :, pl.ds(off, CH)], buf_v.at[p, g],
                    sems[p],
                )

        @pl.when(t >= N_MAIN)
        def _():
            off = (t - N_MAIN) * CH
            for g in range(8):
                pltpu.async_copy(
                    wtl3_hbm.at[g, :, pl.ds(off, CH)], buf_v.at[p, g],
                    sems[p],
                )

    def wait_window(p):
        for g in range(8):
            pltpu.make_async_copy(
                wt3_hbm.at[0, :, pl.ds(0, CH)], buf_v.at[p, 0], sems[p]
            ).wait()

    def _flush(h, pending, sem):
        bv = bext_v[pl.ds(h * 16, 16)]

        @pl.when(pending == 1)
        def _():
            pltpu.make_async_copy(
                ext_v.at[pl.ds(h * 16, 16)], out_hbm.at[bv], sem
            ).wait()

        pltpu.async_copy(ext_v.at[pl.ds(h * 16, 16)], out_hbm.at[bv], sem)

    def scan_window(t, p, state):
        rlo = t * CH

        def grp(q, st):
            slot, f0, f1 = st
            hv = hits_v[pl.ds(q * 16, 16)]
            rel = lax.shift_right_logical(hv, 14)
            bv = hv & 16383
            m = (rel >= rlo) & (rel < rlo + CH) & ((q * 16 + iot) < n_hits)
            mi = jnp.where(m, 1, 0)
            cnt = plsc.all_reduce_population_count(m)[0]

            @pl.when(cnt > 0)
            def _():
                pre = jnp.cumsum(mi) - mi
                for l in range(16):
                    @pl.when(mi[l] == 1)
                    def _():
                        slot_l = (slot + pre[l]) & 31
                        row_v = iot * 0 + slot_l
                        lrel_v = iot * 0 + (rel[l] - rlo)
                        for jq in range(4):
                            vals = plsc.load_gather(
                                buf_v.at[p],
                                [2 * jq + (iot >> 3), iot & 7, lrel_v],
                            )
                            plsc.store_scatter(
                                ext_v, [row_v, jq * 16 + iot], vals
                            )
                        plsc.store_scatter(
                            bext_v, [row_v], iot * 0 + bv[l], mask=lane0
                        )
                        fills = (slot_l & 15) == 15
                        half_l = (slot_l >> 4) & 1

                        @pl.when(fills & (half_l == 0))
                        def _():
                            _flush(0, f0, sc_sem0)

                        @pl.when(fills & (half_l == 1))
                        def _():
                            _flush(1, f1, sc_sem1)

            crossed = ((slot & 15) + cnt) >= 16
            half_c = (slot >> 4) & 1
            f0 = jnp.where(crossed & (half_c == 0), 1, f0)
            f1 = jnp.where(crossed & (half_c == 1), 1, f1)
            slot = (slot + cnt) & 31
            return slot, f0, f1

        return lax.fori_loop(0, n_hit_grp, grp, state)

    fire_window(0, 0)
    fire_window(1, 1)

    def outer(k, state):
        for p in range(2):
            t = 2 * k + p
            wait_window(p)
            state = scan_window(t, p, state)

            @pl.when(t + 2 < N_WIN)
            def _():
                fire_window(t + 2, p)

        return state

    state = lax.fori_loop(0, (N_WIN - 1) // 2, outer, (0, 0, 0))
    wait_window(0)
    slot, f0, f1 = scan_window(N_WIN - 1, 0, state)

    rem = slot & 15
    src_row = (slot - 1) & 31
    src_v = iot * 0 + src_row
    pb = plsc.load_gather(bext_v, [src_v])

    for r in range(1, 16):
        @pl.when((rem != 0) & (r >= rem))
        def _():
            row_v = iot * 0 + ((slot & 16) + r)
            for jq in range(8):
                vals = plsc.load_gather(ext_v, [src_v, jq * 16 + iot])
                plsc.store_scatter(ext_v, [row_v, jq * 16 + iot], vals)
            plsc.store_scatter(bext_v, [row_v], pb, mask=lane0)

    half = (slot >> 4) & 1

    @pl.when((rem != 0) & (half == 0))
    def _():
        _flush(0, f0, sc_sem0)

    @pl.when((rem != 0) & (half == 1))
    def _():
        _flush(1, f1, sc_sem1)

    f0 = jnp.where((rem != 0) & (half == 0), 1, f0)
    f1 = jnp.where((rem != 0) & (half == 1), 1, f1)

    @pl.when(f0 == 1)
    def _():
        bv = bext_v[pl.ds(0, 16)]
        pltpu.make_async_copy(
            ext_v.at[pl.ds(0, 16)], out_hbm.at[bv], sc_sem0
        ).wait()

    @pl.when(f1 == 1)
    def _():
        bv = bext_v[pl.ds(16, 16)]
        pltpu.make_async_copy(
            ext_v.at[pl.ds(16, 16)], out_hbm.at[bv], sc_sem1
        ).wait()


def kernel(x, weight):
    wt3 = weight.T.reshape(8, 8, VOCAB)
    wtail = jnp.pad(weight[MAIN_V:].T, ((0, 0), (0, 2 * CH - TAIL)))
    wtl3 = wtail.reshape(8, 8, 2 * CH)
    out128 = _embed(x.astype(jnp.int32), wt3, wtl3)
    return out128[:, :DIM]


# speedup vs baseline: 21.1477x; 1.6853x over previous
import functools

import jax
import jax.numpy as jnp
from jax import lax
from jax.experimental import pallas as pl
from jax.experimental.pallas import tpu as pltpu
from jax.experimental.pallas import tpu_sc as plsc

BATCH = 16384
DIM = 64
VOCAB = 1000000
MAIN_V = 999424
STRIPE = MAIN_V // 32
TAIL = VOCAB - MAIN_V
CH = 512
N_MAIN = STRIPE // CH
N_WIN = N_MAIN + 2
N_GRP = BATCH // 16

_info = plsc.get_sparse_core_info()
_NC, _NS = _info.num_cores, _info.num_subcores

_mesh = plsc.VectorSubcoreMesh(core_axis_name="c", subcore_axis_name="s")


@functools.partial(
    pl.kernel,
    mesh=_mesh,
    out_type=jax.ShapeDtypeStruct((BATCH, 128), jnp.float32),
    scratch_types=[
        pltpu.VMEM((BATCH + 16,), jnp.int32),
        pltpu.VMEM((BATCH + 16,), jnp.int32),
        pltpu.VMEM((BATCH + 16,), jnp.int32),
        pltpu.VMEM((80,), jnp.int32),
        pltpu.VMEM((64,), jnp.int32),
        pltpu.VMEM((2, 8, 8, CH), jnp.float32),
        pltpu.VMEM((32, 128), jnp.float32),
        pltpu.VMEM((32,), jnp.int32),
        pltpu.SemaphoreType.DMA,
        pltpu.SemaphoreType.DMA,
        pltpu.SemaphoreType.DMA,
        pltpu.SemaphoreType.DMA,
    ],
    compiler_params=pltpu.CompilerParams(needs_layout_passes=False),
)
def _embed(idx_hbm, wt3_hbm, wtl3_hbm, out_hbm, idx_v, hits_v, sort_v,
           st_v, cur_v, buf_v, ext_v, bext_v,
           st_sem0, st_sem1, sc_sem0, sc_sem1):
    wid = lax.axis_index("s") * _NC + lax.axis_index("c")
    lo = wid * STRIPE
    iot = lax.iota(jnp.int32, 16)
    lane0 = iot == 0

    pltpu.sync_copy(idx_hbm.at[pl.ds(0, BATCH)], idx_v.at[pl.ds(0, BATCH)])

    def prepass(q, cur):
        iv = idx_v[pl.ds(q * 16, 16)]
        bv = q * 16 + iot
        m_main = (iv >= lo) & (iv < lo + STRIPE)
        m_tail = (iv >= MAIN_V) & ((bv & 31) == wid)
        rel = jnp.where(m_tail, iv - MAIN_V + STRIPE, iv - lo)
        m = m_main | m_tail
        packed = (rel << 14) | bv
        mi = jnp.where(m, 1, 0)
        csum = jnp.cumsum(mi)
        pos = cur + csum - mi
        plsc.store_scatter(hits_v, [pos], packed, mask=m)
        return cur + csum[15]

    n_hits = lax.fori_loop(0, N_GRP, prepass, 0)
    n_hit_grp = (n_hits + 15) >> 4

    zeros = iot * 0
    for c in range(5):
        st_v[pl.ds(c * 16, 16)] = zeros

    def hist(q, _):
        hv = hits_v[pl.ds(q * 16, 16)]
        w_v = lax.shift_right_logical(hv, 23) & 63
        valid = (q * 16 + iot) < n_hits
        plsc.addupdate_scatter(st_v, [w_v], jnp.where(valid, 1, 0),
                               mask=valid)
        return 0

    lax.fori_loop(0, n_hit_grp, hist, 0)

    carry = 0
    for c in range(4):
        h = st_v[pl.ds(c * 16, 16)]
        cs = jnp.cumsum(h)
        ex = cs - h + carry
        st_v[pl.ds(c * 16, 16)] = ex
        cur_v[pl.ds(c * 16, 16)] = ex
        carry = carry + cs[15]
    st_v[pl.ds(64, 16)] = zeros + carry

    def place(q, _):
        hv = hits_v[pl.ds(q * 16, 16)]
        w_v = lax.shift_right_logical(hv, 23) & 63
        valid = (q * 16 + iot) < n_hits
        vi = jnp.where(valid, 1, 0)
        for l in range(16):
            @pl.when(vi[l] == 1)
            def _():
                wsp = iot * 0 + w_v[l]
                cur = plsc.load_gather(cur_v, [wsp])
                plsc.store_scatter(sort_v, [cur], iot * 0 + hv[l],
                                   mask=lane0)
                plsc.store_scatter(cur_v, [wsp], cur + 1, mask=lane0)
        return 0

    lax.fori_loop(0, n_hit_grp, place, 0)

    def get_start(w):
        chunk = st_v[pl.ds((w >> 4) * 16, 16)]
        return jnp.sum(jnp.where(iot == (w & 15), chunk, 0))

    sems = (st_sem0, st_sem1)

    def fire_window(t, p):
        @pl.when(t < N_MAIN)
        def _():
            off = lo + t * CH
            for g in range(8):
                pltpu.async_copy(
                    wt3_hbm.at[g, :, pl.ds(off, CH)], buf_v.at[p, g],
                    sems[p],
                )

        @pl.when(t >= N_MAIN)
        def _():
            off = (t - N_MAIN) * CH
            for g in range(8):
                pltpu.async_copy(
                    wtl3_hbm.at[g, :, pl.ds(off, CH)], buf_v.at[p, g],
                    sems[p],
                )

    def wait_window(p):
        for g in range(8):
            pltpu.make_async_copy(
                wt3_hbm.at[0, :, pl.ds(0, CH)], buf_v.at[p, 0], sems[p]
            ).wait()

    def _flush(h, pending, sem):
        bv = bext_v[pl.ds(h * 16, 16)]

        @pl.when(pending == 1)
        def _():
            pltpu.make_async_copy(
                ext_v.at[pl.ds(h * 16, 16)], out_hbm.at[bv], sem
            ).wait()

        pltpu.async_copy(ext_v.at[pl.ds(h * 16, 16)], out_hbm.at[bv], sem)

    def scan_window(t, p, state):
        f0, f1, nf = state
        start = get_start(t)
        end = get_start(t + 1)
        rlo = t * CH

        def grp(g, _):
            posv = g * 16 + iot
            hv = sort_v[pl.ds(g * 16, 16)]
            m = (posv >= start) & (posv < end)
            rel = lax.shift_right_logical(hv, 14)
            bv = hv & 16383
            lrel = (rel - rlo) & (CH - 1)
            row_v = posv & 31
            for j in range(64):
                vals = plsc.load_gather(
                    buf_v.at[p],
                    [zeros + (j >> 3), zeros + (j & 7), lrel],
                    mask=m,
                )
                plsc.store_scatter(ext_v, [row_v, zeros + j], vals, mask=m)
            plsc.store_scatter(bext_v, [row_v], bv, mask=m)
            return 0

        lax.fori_loop(start >> 4, (end + 15) >> 4, grp, 0)

        def flush_blk(i, st2):
            f0, f1 = st2

            @pl.when((i & 1) == 0)
            def _():
                _flush(0, f0, sc_sem0)

            @pl.when((i & 1) == 1)
            def _():
                _flush(1, f1, sc_sem1)

            f0 = jnp.where((i & 1) == 0, 1, f0)
            f1 = jnp.where((i & 1) == 1, 1, f1)
            return f0, f1

        nf_new = end >> 4
        f0, f1 = lax.fori_loop(nf, nf_new, flush_blk, (f0, f1))
        return f0, f1, nf_new

    fire_window(0, 0)
    fire_window(1, 1)

    def outer(k, state):
        for p in range(2):
            t = 2 * k + p
            wait_window(p)
            state = scan_window(t, p, state)

            @pl.when(t + 2 < N_WIN)
            def _():
                fire_window(t + 2, p)

        return state

    state = lax.fori_loop(0, (N_WIN - 1) // 2, outer, (0, 0, 0))
    wait_window(0)
    f0, f1, nf = scan_window(N_WIN - 1, 0, state)

    slot = n_hits & 31
    rem = n_hits & 15
    src_row = (n_hits - 1) & 31
    src_v = iot * 0 + src_row
    pb = plsc.load_gather(bext_v, [src_v])

    for r in range(1, 16):
        @pl.when((rem != 0) & (r >= rem))
        def _():
            row_v = iot * 0 + ((slot & 16) + r)
            for jq in range(8):
                vals = plsc.load_gather(ext_v, [src_v, jq * 16 + iot])
                plsc.store_scatter(ext_v, [row_v, jq * 16 + iot], vals)
            plsc.store_scatter(bext_v, [row_v], pb, mask=lane0)

    half = (slot >> 4) & 1

    @pl.when((rem != 0) & (half == 0))
    def _():
        _flush(0, f0, sc_sem0)

    @pl.when((rem != 0) & (half == 1))
    def _():
        _flush(1, f1, sc_sem1)

    f0 = jnp.where((rem != 0) & (half == 0), 1, f0)
    f1 = jnp.where((rem != 0) & (half == 1), 1, f1)

    @pl.when(f0 == 1)
    def _():
        bv = bext_v[pl.ds(0, 16)]
        pltpu.make_async_copy(
            ext_v.at[pl.ds(0, 16)], out_hbm.at[bv], sc_sem0
        ).wait()

    @pl.when(f1 == 1)
    def _():
        bv = bext_v[pl.ds(16, 16)]
        pltpu.make_async_copy(
            ext_v.at[pl.ds(16, 16)], out_hbm.at[bv], sc_sem1
        ).wait()


def kernel(x, weight):
    wt3 = weight.T.reshape(8, 8, VOCAB)
    wtail = jnp.pad(weight[MAIN_V:].T, ((0, 0), (0, 2 * CH - TAIL)))
    wtl3 = wtail.reshape(8, 8, 2 * CH)
    out128 = _embed(x.astype(jnp.int32), wt3, wtl3)
    return out128[:, :DIM]
